# baseline (speedup 1.0000x reference)
import jax
import jax.numpy as jnp
from jax.experimental import pallas as pl
from jax.experimental.pallas import tpu as pltpu

_BM = 1024
_MT = 4096 // _BM
_NSTAGE = 20


def _ds(i, width):
    return pl.ds(pl.multiple_of(i * width, width), width)


def _fused_kernel(x_ref, w1_ref, w2_ref, w3_ref, b_ref,
                  o_ref, h1_ref, h2_ref):
    s = pl.program_id(1)

    @pl.when(s < 8)
    def _l1():
        xb = x_ref[...].astype(jnp.bfloat16)
        wb = w1_ref[...].astype(jnp.bfloat16)
        acc = jnp.dot(xb, wb, preferred_element_type=jnp.float32)
        acc = jnp.maximum(acc + b_ref[0:1, _ds(s, 512)], 0.0)
        h1_ref[:, _ds(s, 512)] = acc.astype(jnp.bfloat16)

    @pl.when((s >= 8) & (s < 16))
    def _l2():
        n = s - 8
        wb = w2_ref[...].astype(jnp.bfloat16)
        acc = jnp.dot(h1_ref[...], wb, preferred_element_type=jnp.float32)
        acc = jnp.maximum(acc + b_ref[1:2, _ds(n, 512)], 0.0)
        h2_ref[:, _ds(n, 512)] = acc.astype(jnp.bfloat16)

    @pl.when(s >= 16)
    def _l3():
        k = s - 16
        wb = w3_ref[...].astype(jnp.bfloat16)
        acc = jnp.dot(h2_ref[:, _ds(k, 1024)], wb,
                      preferred_element_type=jnp.float32)

        @pl.when(s == 16)
        def _init():
            o_ref[...] = acc + b_ref[2:3, :1000]

        @pl.when(s > 16)
        def _accum():
            o_ref[...] += acc


def kernel(x, W1, b1, W2, b2, W3, b3, interpret=False):
    ball = jnp.stack([b1, b2,
                      jnp.pad(b3, (0, 4096 - 1000))], axis=0)
    return pl.pallas_call(
        _fused_kernel,
        grid=(_MT, _NSTAGE),
        in_specs=[
            pl.BlockSpec((_BM, 1024), lambda m, s: (m, 0)),
            pl.BlockSpec((1024, 512),
                         lambda m, s: (0, jnp.minimum(s, 7))),
            pl.BlockSpec((4096, 512),
                         lambda m, s: (0, jnp.clip(s - 8, 0, 7))),
            pl.BlockSpec((1024, 1000),
                         lambda m, s: (jnp.clip(s - 16, 0, 3), 0)),
            pl.BlockSpec((3, 4096), lambda m, s: (0, 0)),
        ],
        out_specs=pl.BlockSpec((_BM, 1000), lambda m, s: (m, 0)),
        out_shape=jax.ShapeDtypeStruct((4096, 1000), jnp.float32),
        scratch_shapes=[
            pltpu.VMEM((_BM, 4096), jnp.bfloat16),
            pltpu.VMEM((_BM, 4096), jnp.bfloat16),
        ],
        compiler_params=pltpu.CompilerParams(
            dimension_semantics=("arbitrary", "arbitrary"),
            vmem_limit_bytes=64 * 1024 * 1024,
        ),
        interpret=interpret,
    )(x, W1, W2, W3, ball)


# speedup vs baseline: 1.0504x; 1.0504x over previous
import functools

import jax
import jax.numpy as jnp
from jax.experimental import pallas as pl
from jax.experimental.pallas import tpu as pltpu


def _mm_kernel(x_ref, w_ref, b_ref, o_ref, *, act, out_dtype):
    x = x_ref[...].astype(jnp.bfloat16)
    w = w_ref[...].astype(jnp.bfloat16)
    acc = jnp.dot(x, w, preferred_element_type=jnp.float32)
    acc = acc + b_ref[...]
    if act:
        acc = jnp.maximum(acc, 0.0)
    o_ref[...] = acc.astype(out_dtype)


def _layer(h, w, b, *, bm, bn, act, out_dtype, interpret=False):
    M, K = h.shape
    _, N = w.shape
    n_tiles = N // bn
    m_tiles = M // bm
    body = functools.partial(_mm_kernel, act=act, out_dtype=out_dtype)
    return pl.pallas_call(
        body,
        grid=(n_tiles, m_tiles),
        in_specs=[
            pl.BlockSpec((bm, K), lambda n, m: (m, 0)),
            pl.BlockSpec((K, bn), lambda n, m: (0, n)),
            pl.BlockSpec((1, bn), lambda n, m: (0, n)),
        ],
        out_specs=pl.BlockSpec((bm, bn), lambda n, m: (m, n)),
        out_shape=jax.ShapeDtypeStruct((M, N), out_dtype),
        compiler_params=pltpu.CompilerParams(
            dimension_semantics=("arbitrary", "arbitrary"),
        ),
        interpret=interpret,
    )(h, w, b)


def kernel(x, W1, b1, W2, b2, W3, b3, interpret=False):
    h1 = _layer(x, W1, b1.reshape(1, -1), bm=512, bn=4096, act=True,
                out_dtype=jnp.bfloat16, interpret=interpret)
    h2 = _layer(h1, W2, b2.reshape(1, -1), bm=512, bn=1024, act=True,
                out_dtype=jnp.bfloat16, interpret=interpret)
    out = _layer(h2, W3, b3.reshape(1, -1), bm=512, bn=1000, act=False,
                 out_dtype=jnp.float32, interpret=interpret)
    return out


# speedup vs baseline: 1.0885x; 1.0363x over previous
import functools

import jax
import jax.numpy as jnp
from jax.experimental import pallas as pl
from jax.experimental.pallas import tpu as pltpu


def _mm_kernel(x_ref, w_ref, b_ref, o_ref, *, act, out_dtype):
    x = x_ref[...].astype(jnp.bfloat16)
    w = w_ref[...].astype(jnp.bfloat16)
    acc = jnp.dot(x, w, preferred_element_type=jnp.float32)
    acc = acc + b_ref[...]
    if act:
        acc = jnp.maximum(acc, 0.0)
    o_ref[...] = acc.astype(out_dtype)


def _layer(h, w, b, *, bm, bn, act, out_dtype, interpret=False):
    M, K = h.shape
    _, N = w.shape
    n_tiles = N // bn
    m_tiles = M // bm
    body = functools.partial(_mm_kernel, act=act, out_dtype=out_dtype)
    return pl.pallas_call(
        body,
        grid=(n_tiles, m_tiles),
        in_specs=[
            pl.BlockSpec((bm, K), lambda n, m: (m, 0)),
            pl.BlockSpec((K, bn), lambda n, m: (0, n)),
            pl.BlockSpec((1, bn), lambda n, m: (0, n)),
        ],
        out_specs=pl.BlockSpec((bm, bn), lambda n, m: (m, n)),
        out_shape=jax.ShapeDtypeStruct((M, N), out_dtype),
        compiler_params=pltpu.CompilerParams(
            dimension_semantics=("arbitrary", "arbitrary"),
            vmem_limit_bytes=64 * 1024 * 1024,
        ),
        interpret=interpret,
    )(h, w, b)


def kernel(x, W1, b1, W2, b2, W3, b3, interpret=False):
    h1 = _layer(x, W1, b1.reshape(1, -1), bm=512, bn=4096, act=True,
                out_dtype=jnp.bfloat16, interpret=interpret)
    h2 = _layer(h1, W2, b2.reshape(1, -1), bm=1024, bn=1024, act=True,
                out_dtype=jnp.bfloat16, interpret=interpret)
    out = _layer(h2, W3, b3.reshape(1, -1), bm=1024, bn=1000, act=False,
                 out_dtype=jnp.float32, interpret=interpret)
    return out
